# Initial kernel scaffold; baseline (speedup 1.0000x reference)
import jax
import jax.numpy as jnp
from jax.experimental import pallas as pl


def kernel(pos_edge_index, neg_edge_index, num_nodes):
    raise NotImplementedError("write your pallas kernel here")


# speedup vs baseline: 3.7223x; 3.7223x over previous
import functools

import jax
import jax.numpy as jnp
from jax import lax
from jax.experimental import pallas as pl
from jax.experimental.pallas import tpu as pltpu
from jax.experimental.pallas import tpu_sc as plsc

N = 10000
E = 320000
LANES = 16
CHUNK = 128
NCHUNKS = E // CHUNK
NWORKERS = 32
RPW = 79
EPW = RPW * CHUNK

_mesh = plsc.VectorSubcoreMesh(core_axis_name="c", subcore_axis_name="s")


@functools.partial(
    pl.kernel,
    out_type=(),
    mesh=_mesh,
    scratch_types=[
        pltpu.VMEM((EPW,), jnp.int32),
        pltpu.VMEM((EPW,), jnp.int32),
        pltpu.VMEM((EPW,), jnp.int32),
        pltpu.VMEM((EPW,), jnp.int32),
        pltpu.VMEM((EPW,), jnp.float32),
        pltpu.SemaphoreType.DMA,
    ],
)
def _scatter_edges(adj_ref, e0_hbm, e1_hbm, vals_hbm,
                   e0_v, e1_v, idx1_v, idx2_v, vals_v, ssem):
    w = lax.axis_index("s") * 2 + lax.axis_index("c")
    start_chunk = jnp.minimum(w * NCHUNKS // NWORKERS, NCHUNKS - RPW)
    base = start_chunk * CHUNK
    pltpu.sync_copy(e0_hbm.at[pl.ds(base, EPW)], e0_v)
    pltpu.sync_copy(e1_hbm.at[pl.ds(base, EPW)], e1_v)
    pltpu.sync_copy(vals_hbm, vals_v)

    def compute(r, carry):
        for k in range(CHUNK // LANES):
            off = r * CHUNK + k * LANES
            a = e0_v[pl.ds(off, LANES)]
            b = e1_v[pl.ds(off, LANES)]
            idx1_v[pl.ds(off, LANES)] = a * N + b
            idx2_v[pl.ds(off, LANES)] = b * N + a
        return carry

    lax.fori_loop(0, RPW, compute, 0)
    pltpu.async_copy(vals_v, adj_ref.at[idx1_v], ssem)
    pltpu.async_copy(vals_v, adj_ref.at[idx2_v], ssem)
    pltpu.make_async_copy(vals_v, adj_ref.at[idx1_v], ssem).wait()
    pltpu.make_async_copy(vals_v, adj_ref.at[idx2_v], ssem).wait()


_ROWS_PER_BLK = 240


def _norm_body(x_ref, o_ref):
    x = x_ref[...]
    s = jnp.sum(x, axis=1, keepdims=True)
    o_ref[...] = x / (s + 1e-10)


_normalize = pl.pallas_call(
    _norm_body,
    grid=(pl.cdiv(N, _ROWS_PER_BLK),),
    in_specs=[pl.BlockSpec((_ROWS_PER_BLK, N), lambda b: (b, 0))],
    out_specs=pl.BlockSpec((_ROWS_PER_BLK, N), lambda b: (b, 0)),
    out_shape=jax.ShapeDtypeStruct((N, N), jnp.float32),
)


def kernel(pos_edge_index, neg_edge_index, num_nodes):
    del num_nodes
    adj_ref = jax.new_ref(jnp.zeros((N * N,), jnp.float32))
    ones = jnp.ones((EPW,), jnp.float32)
    _scatter_edges(adj_ref, pos_edge_index[0], pos_edge_index[1], ones)
    _scatter_edges(adj_ref, neg_edge_index[0], neg_edge_index[1], -ones)
    adj = jax.freeze(adj_ref).reshape(N, N)
    return _normalize(adj)


# speedup vs baseline: 3.7223x; 1.0000x over previous
import functools

import jax
import jax.numpy as jnp
from jax import lax
from jax.experimental import pallas as pl
from jax.experimental.pallas import tpu as pltpu
from jax.experimental.pallas import tpu_sc as plsc

N = 10000
E = 320000
LANES = 16
CHUNK = 128
NCHUNKS = E // CHUNK
NWORKERS = 32
RPW = 79
EPW = RPW * CHUNK

_mesh = plsc.VectorSubcoreMesh(core_axis_name="c", subcore_axis_name="s")


@functools.partial(
    pl.kernel,
    out_type=(),
    mesh=_mesh,
    scratch_types=[
        pltpu.VMEM((EPW,), jnp.int32),
        pltpu.VMEM((EPW,), jnp.int32),
        pltpu.VMEM((EPW,), jnp.int32),
        pltpu.VMEM((EPW,), jnp.int32),
        pltpu.VMEM((EPW,), jnp.float32),
        pltpu.SemaphoreType.DMA,
        pltpu.SemaphoreType.DMA,
    ],
)
def _scatter_edges(adj_ref, e0_hbm, e1_hbm, vals_hbm,
                   e0_v, e1_v, idx1_v, idx2_v, vals_v, ssem, ssem2):
    w = lax.axis_index("s") * 2 + lax.axis_index("c")
    start_chunk = jnp.minimum(w * NCHUNKS // NWORKERS, NCHUNKS - RPW)
    base = start_chunk * CHUNK
    pltpu.sync_copy(e0_hbm.at[pl.ds(base, EPW)], e0_v)
    pltpu.sync_copy(e1_hbm.at[pl.ds(base, EPW)], e1_v)
    pltpu.sync_copy(vals_hbm, vals_v)

    def compute(r, carry):
        for k in range(CHUNK // LANES):
            off = r * CHUNK + k * LANES
            a = e0_v[pl.ds(off, LANES)]
            b = e1_v[pl.ds(off, LANES)]
            idx1_v[pl.ds(off, LANES)] = a * N + b
            idx2_v[pl.ds(off, LANES)] = b * N + a
        return carry

    lax.fori_loop(0, RPW, compute, 0)
    pltpu.async_copy(vals_v, adj_ref.at[idx1_v], ssem)
    pltpu.async_copy(vals_v, adj_ref.at[idx2_v], ssem2)
    pltpu.make_async_copy(vals_v, adj_ref.at[idx1_v], ssem).wait()
    pltpu.make_async_copy(vals_v, adj_ref.at[idx2_v], ssem2).wait()


_ROWS_PER_BLK = 200


def _norm_body(x_ref, o_ref):
    x = x_ref[...]
    s = jnp.sum(x, axis=1, keepdims=True)
    o_ref[...] = x / (s + 1e-10)


_normalize = pl.pallas_call(
    _norm_body,
    grid=(pl.cdiv(N, _ROWS_PER_BLK),),
    in_specs=[pl.BlockSpec((_ROWS_PER_BLK, N), lambda b: (b, 0))],
    out_specs=pl.BlockSpec((_ROWS_PER_BLK, N), lambda b: (b, 0)),
    out_shape=jax.ShapeDtypeStruct((N, N), jnp.float32),
)


def kernel(pos_edge_index, neg_edge_index, num_nodes):
    del num_nodes
    adj_ref = jax.new_ref(jnp.zeros((N * N,), jnp.float32))
    ones = jnp.ones((EPW,), jnp.float32)
    _scatter_edges(adj_ref, pos_edge_index[0], pos_edge_index[1], ones)
    _scatter_edges(adj_ref, neg_edge_index[0], neg_edge_index[1], -ones)
    adj = jax.freeze(adj_ref).reshape(N, N)
    return _normalize(adj)


# speedup vs baseline: 3.7226x; 1.0001x over previous
import functools

import jax
import jax.numpy as jnp
from jax import lax
from jax.experimental import pallas as pl
from jax.experimental.pallas import tpu as pltpu
from jax.experimental.pallas import tpu_sc as plsc

N = 10000
E = 320000
LANES = 16
CHUNK = 128
NCHUNKS = E // CHUNK
NWORKERS = 32
RPW = 79
EPW = RPW * CHUNK

_mesh = plsc.VectorSubcoreMesh(core_axis_name="c", subcore_axis_name="s")


@functools.partial(
    pl.kernel,
    out_type=(),
    mesh=_mesh,
    scratch_types=[
        pltpu.VMEM((EPW,), jnp.int32),
        pltpu.VMEM((EPW,), jnp.int32),
        pltpu.VMEM((EPW,), jnp.int32),
        pltpu.VMEM((EPW,), jnp.int32),
        pltpu.VMEM((EPW,), jnp.float32),
        pltpu.SemaphoreType.DMA,
    ],
)
def _scatter_edges(adj_ref, e0_hbm, e1_hbm, vals_hbm,
                   e0_v, e1_v, idx1_v, idx2_v, vals_v, ssem):
    w = lax.axis_index("s") * 2 + lax.axis_index("c")
    start_chunk = jnp.minimum(w * NCHUNKS // NWORKERS, NCHUNKS - RPW)
    base = start_chunk * CHUNK
    pltpu.sync_copy(e0_hbm.at[pl.ds(base, EPW)], e0_v)
    pltpu.sync_copy(e1_hbm.at[pl.ds(base, EPW)], e1_v)
    pltpu.sync_copy(vals_hbm, vals_v)

    def compute(r, carry):
        for k in range(CHUNK // LANES):
            off = r * CHUNK + k * LANES
            a = e0_v[pl.ds(off, LANES)]
            b = e1_v[pl.ds(off, LANES)]
            idx1_v[pl.ds(off, LANES)] = a * N + b
            idx2_v[pl.ds(off, LANES)] = b * N + a
        return carry

    lax.fori_loop(0, RPW, compute, 0)
    pltpu.async_copy(vals_v, adj_ref.at[idx1_v], ssem)
    pltpu.async_copy(vals_v, adj_ref.at[idx2_v], ssem)
    pltpu.make_async_copy(vals_v, adj_ref.at[idx1_v], ssem).wait()
    pltpu.make_async_copy(vals_v, adj_ref.at[idx2_v], ssem).wait()


_ROWS_PER_BLK = 200


def _norm_body(x_ref, o_ref):
    x = x_ref[...]
    s = jnp.sum(x, axis=1, keepdims=True)
    o_ref[...] = x / (s + 1e-10)


_normalize = pl.pallas_call(
    _norm_body,
    grid=(pl.cdiv(N, _ROWS_PER_BLK),),
    in_specs=[pl.BlockSpec((_ROWS_PER_BLK, N), lambda b: (b, 0))],
    out_specs=pl.BlockSpec((_ROWS_PER_BLK, N), lambda b: (b, 0)),
    out_shape=jax.ShapeDtypeStruct((N, N), jnp.float32),
)


def kernel(pos_edge_index, neg_edge_index, num_nodes):
    del num_nodes
    adj_ref = jax.new_ref(jnp.zeros((N * N,), jnp.float32))
    ones = jnp.ones((EPW,), jnp.float32)
    _scatter_edges(adj_ref, pos_edge_index[0], pos_edge_index[1], ones)
    _scatter_edges(adj_ref, neg_edge_index[0], neg_edge_index[1], -ones)
    adj = jax.freeze(adj_ref).reshape(N, N)
    return _normalize(adj)
